# baseline (speedup 1.0000x reference)
import functools

import jax
import jax.numpy as jnp
from jax import lax
from jax.experimental import pallas as pl
from jax.experimental.pallas import tpu as pltpu
from jax.experimental.pallas import tpu_sc as plsc

ALIGN = 8
BLK = 64
MBLK = 128
NC = 2
NS = 16
NW = NC * NS
WLANES = 128
RING = 5
AHEAD = 4


def _meta_body(e_ref, w_ref, inv_ref, off_ref, nblk_ref, wrow_ref):
    n_tok = e_ref.shape[0]
    n_exp = off_ref.shape[1]
    ecol = e_ref[:]
    onehot = (
        ecol == lax.broadcasted_iota(jnp.int32, (n_tok, n_exp), 1)
    ).astype(jnp.float32)

    counts = jnp.sum(onehot, axis=0, keepdims=True).astype(jnp.int32)
    counts_a = ((counts + ALIGN - 1) // ALIGN) * ALIGN
    lt_e = (
        lax.broadcasted_iota(jnp.int32, (n_exp, n_exp), 0)
        < lax.broadcasted_iota(jnp.int32, (n_exp, n_exp), 1)
    ).astype(jnp.float32)
    off_f = jnp.dot(
        counts_a.astype(jnp.float32), lt_e, preferred_element_type=jnp.float32
    )

    tri = (
        lax.broadcasted_iota(jnp.int32, (MBLK, MBLK), 0)
        > lax.broadcasted_iota(jnp.int32, (MBLK, MBLK), 1)
    ).astype(jnp.float32)

    carry = jnp.zeros((1, n_exp), jnp.float32)
    for b in range(n_tok // MBLK):
        ob = onehot[b * MBLK : (b + 1) * MBLK, :]
        cb = jnp.dot(tri, ob, preferred_element_type=jnp.float32) + carry
        rank_b = jnp.sum(cb * ob, axis=1, keepdims=True)
        base_b = jnp.sum(off_f * ob, axis=1, keepdims=True)
        inv_ref[b * MBLK : (b + 1) * MBLK, :] = (rank_b + base_b).astype(jnp.int32)
        carry = carry + jnp.sum(ob, axis=0, keepdims=True)

    off_ref[...] = off_f.astype(jnp.int32)
    nblk_ref[...] = (counts_a + BLK - 1) // BLK
    wrow_ref[...] = jnp.broadcast_to(w_ref[:], (n_tok, WLANES))


def _ffn_body(
    off_ref, nblk_ref, xs_ref, ws_ref, w1_ref, w2_ref, w3_ref, out_ref,
    wb1, wb2, wb3, sem1, sem2, sem3,
):
    e = pl.program_id(0)
    n_exp = pl.num_programs(0)

    def issue(expert, slot):
        pltpu.make_async_copy(w1_ref.at[expert], wb1.at[slot], sem1.at[slot]).start()
        pltpu.make_async_copy(w2_ref.at[expert], wb2.at[slot], sem2.at[slot]).start()
        pltpu.make_async_copy(w3_ref.at[expert], wb3.at[slot], sem3.at[slot]).start()

    @pl.when(e == 0)
    def _prime():
        for k in range(AHEAD):
            issue(k, k)

    @pl.when(e + AHEAD < n_exp)
    def _ahead():
        issue(e + AHEAD, lax.rem(e + AHEAD, RING))

    slot = lax.rem(e, RING)
    pltpu.make_async_copy(w1_ref.at[e], wb1.at[slot], sem1.at[slot]).wait()
    pltpu.make_async_copy(w2_ref.at[e], wb2.at[slot], sem2.at[slot]).wait()
    pltpu.make_async_copy(w3_ref.at[e], wb3.at[slot], sem3.at[slot]).wait()

    start = off_ref[0, e]
    nblk = nblk_ref[0, e]
    wa = wb1[slot]
    wb = wb2[slot]
    wc = wb3[slot]

    def body(ib, carry):
        base = pl.multiple_of(start + ib * BLK, ALIGN)
        xb = xs_ref[pl.ds(base, BLK), :]
        g = jnp.dot(xb, wa, preferred_element_type=jnp.float32)
        v = jnp.dot(xb, wb, preferred_element_type=jnp.float32)
        h = (g * jax.nn.sigmoid(g)) * v
        o = jnp.dot(h, wc, preferred_element_type=jnp.float32)
        o = o * ws_ref[pl.ds(base, BLK), :][:, 0:1]
        out_ref[pl.ds(base, BLK), :] = o
        return carry

    lax.fori_loop(0, nblk, body, 0)


def _sc_mesh():
    return plsc.VectorSubcoreMesh(
        core_axis_name="c", subcore_axis_name="s", num_cores=NC, num_subcores=NS
    )


def _sc_dispatch_fn(n_tokens, p_tot, d_model):
    bpw = n_tokens // NW

    @functools.partial(
        pl.kernel,
        out_type=(
            jax.ShapeDtypeStruct((p_tot, d_model), jnp.float32),
            jax.ShapeDtypeStruct((p_tot, WLANES), jnp.float32),
        ),
        mesh=_sc_mesh(),
        scratch_types=[
            pltpu.VMEM((bpw,), jnp.int32),
            pltpu.VMEM((bpw, d_model), jnp.float32),
            pltpu.VMEM((bpw, WLANES), jnp.float32),
            pltpu.SemaphoreType.DMA,
            pltpu.SemaphoreType.DMA,
        ],
    )
    def dispatch(x_hbm, wrow_hbm, inv_hbm, xs_hbm, ws_hbm, idx_v, rows_v, wrows_v, sem_a, sem_b):
        wid = lax.axis_index("s") * NC + lax.axis_index("c")
        base = wid * bpw
        pltpu.sync_copy(inv_hbm.at[pl.ds(base, bpw)], idx_v)
        pltpu.sync_copy(x_hbm.at[pl.ds(base, bpw)], rows_v)
        pltpu.sync_copy(wrow_hbm.at[pl.ds(base, bpw)], wrows_v)
        cp_a = pltpu.async_copy(rows_v, xs_hbm.at[idx_v], sem_a)
        cp_b = pltpu.async_copy(wrows_v, ws_hbm.at[idx_v], sem_b)
        cp_a.wait()
        cp_b.wait()

    return dispatch


def _sc_combine_fn(n_tokens, p_tot, d_model):
    bpw = n_tokens // NW

    @functools.partial(
        pl.kernel,
        out_type=jax.ShapeDtypeStruct((n_tokens, d_model), jnp.float32),
        mesh=_sc_mesh(),
        scratch_types=[
            pltpu.VMEM((bpw,), jnp.int32),
            pltpu.VMEM((bpw, d_model), jnp.float32),
            pltpu.SemaphoreType.DMA,
        ],
    )
    def combine(ys_hbm, inv_hbm, out_hbm, idx_v, rows_v, sem):
        wid = lax.axis_index("s") * NC + lax.axis_index("c")
        base = wid * bpw
        pltpu.sync_copy(inv_hbm.at[pl.ds(base, bpw)], idx_v)
        pltpu.async_copy(ys_hbm.at[idx_v], rows_v, sem).wait()
        pltpu.sync_copy(rows_v, out_hbm.at[pl.ds(base, bpw)])

    return combine


@jax.jit
def kernel(x, expert_indices, expert_weights, w1, w2, w3):
    n_tokens, d_model = x.shape
    n_exp = w1.shape[0]
    d_ff = w1.shape[2]

    p_tot = n_tokens + n_exp * (ALIGN - 1) + BLK
    p_tot = ((p_tot + BLK - 1) // BLK) * BLK

    e_col = expert_indices
    if e_col.dtype != jnp.int32:
        e_col = e_col.astype(jnp.int32)
    if e_col.shape != (n_tokens, 1):
        e_col = e_col.reshape(n_tokens, 1)
    w_col = expert_weights
    if w_col.dtype != jnp.float32:
        w_col = w_col.astype(jnp.float32)
    if w_col.shape != (n_tokens, 1):
        w_col = w_col.reshape(n_tokens, 1)

    inv2d, off2d, nblk2d, wrow = pl.pallas_call(
        _meta_body,
        out_shape=(
            jax.ShapeDtypeStruct((n_tokens, 1), jnp.int32),
            jax.ShapeDtypeStruct((1, n_exp), jnp.int32),
            jax.ShapeDtypeStruct((1, n_exp), jnp.int32),
            jax.ShapeDtypeStruct((n_tokens, WLANES), jnp.float32),
        ),
    )(e_col, w_col)
    inv = inv2d.reshape(n_tokens)

    xs, ws = _sc_dispatch_fn(n_tokens, p_tot, d_model)(x, wrow, inv)

    ys = pl.pallas_call(
        _ffn_body,
        grid=(n_exp,),
        in_specs=[
            pl.BlockSpec(memory_space=pltpu.SMEM),
            pl.BlockSpec(memory_space=pltpu.SMEM),
            pl.BlockSpec((p_tot, d_model), lambda e: (0, 0)),
            pl.BlockSpec((p_tot, WLANES), lambda e: (0, 0)),
            pl.BlockSpec(memory_space=pl.ANY),
            pl.BlockSpec(memory_space=pl.ANY),
            pl.BlockSpec(memory_space=pl.ANY),
        ],
        out_specs=pl.BlockSpec((p_tot, d_model), lambda e: (0, 0)),
        out_shape=jax.ShapeDtypeStruct((p_tot, d_model), jnp.float32),
        scratch_shapes=[
            pltpu.VMEM((RING, d_model, d_ff), jnp.float32),
            pltpu.VMEM((RING, d_model, d_ff), jnp.float32),
            pltpu.VMEM((RING, d_ff, d_model), jnp.float32),
            pltpu.SemaphoreType.DMA((RING,)),
            pltpu.SemaphoreType.DMA((RING,)),
            pltpu.SemaphoreType.DMA((RING,)),
        ],
        compiler_params=pltpu.CompilerParams(
            dimension_semantics=("arbitrary",),
        ),
    )(off2d, nblk2d, xs, ws, w1, w2, w3)

    return _sc_combine_fn(n_tokens, p_tot, d_model)(ys, inv)


# speedup vs baseline: 1.0028x; 1.0028x over previous
import functools

import jax
import jax.numpy as jnp
from jax import lax
from jax.experimental import pallas as pl
from jax.experimental.pallas import tpu as pltpu
from jax.experimental.pallas import tpu_sc as plsc

ALIGN = 8
BLK = 64
MBLK = 128
NC = 2
NS = 16
NW = NC * NS
WLANES = 128
EPG = 2


def _meta_body(e_ref, w_ref, inv_ref, off_ref, nblk_ref, wrow_ref):
    n_tok = e_ref.shape[0]
    n_exp = off_ref.shape[1]
    ecol = e_ref[:]
    onehot = (
        ecol == lax.broadcasted_iota(jnp.int32, (n_tok, n_exp), 1)
    ).astype(jnp.float32)

    counts = jnp.sum(onehot, axis=0, keepdims=True).astype(jnp.int32)
    counts_a = ((counts + ALIGN - 1) // ALIGN) * ALIGN
    lt_e = (
        lax.broadcasted_iota(jnp.int32, (n_exp, n_exp), 0)
        < lax.broadcasted_iota(jnp.int32, (n_exp, n_exp), 1)
    ).astype(jnp.float32)
    off_f = jnp.dot(
        counts_a.astype(jnp.float32), lt_e, preferred_element_type=jnp.float32
    )

    tri = (
        lax.broadcasted_iota(jnp.int32, (MBLK, MBLK), 0)
        > lax.broadcasted_iota(jnp.int32, (MBLK, MBLK), 1)
    ).astype(jnp.float32)

    carry = jnp.zeros((1, n_exp), jnp.float32)
    for b in range(n_tok // MBLK):
        ob = onehot[b * MBLK : (b + 1) * MBLK, :]
        cb = jnp.dot(tri, ob, preferred_element_type=jnp.float32) + carry
        rank_b = jnp.sum(cb * ob, axis=1, keepdims=True)
        base_b = jnp.sum(off_f * ob, axis=1, keepdims=True)
        inv_ref[b * MBLK : (b + 1) * MBLK, :] = (rank_b + base_b).astype(jnp.int32)
        carry = carry + jnp.sum(ob, axis=0, keepdims=True)

    off_ref[...] = off_f.astype(jnp.int32)
    nblk_ref[...] = (counts_a + BLK - 1) // BLK
    wrow_ref[...] = jnp.broadcast_to(w_ref[:], (n_tok, WLANES))


def _ffn_body(off_ref, nblk_ref, xs_ref, ws_ref, w1_ref, w2_ref, w3_ref, out_ref):
    pid = pl.program_id(0)
    for j in range(EPG):
        e = pid * EPG + j
        start = off_ref[0, e]
        nblk = nblk_ref[0, e]
        wa = w1_ref[j]
        wb = w2_ref[j]
        wc = w3_ref[j]

        def body(ib, carry):
            base = pl.multiple_of(start + ib * BLK, ALIGN)
            xb = xs_ref[pl.ds(base, BLK), :]
            g = jnp.dot(xb, wa, preferred_element_type=jnp.float32)
            v = jnp.dot(xb, wb, preferred_element_type=jnp.float32)
            h = (g * jax.nn.sigmoid(g)) * v
            o = jnp.dot(h, wc, preferred_element_type=jnp.float32)
            o = o * ws_ref[pl.ds(base, BLK), :][:, 0:1]
            out_ref[pl.ds(base, BLK), :] = o
            return carry

        lax.fori_loop(0, nblk, body, 0)


def _sc_mesh():
    return plsc.VectorSubcoreMesh(
        core_axis_name="c", subcore_axis_name="s", num_cores=NC, num_subcores=NS
    )


def _sc_dispatch_fn(n_tokens, p_tot, d_model):
    bpw = n_tokens // NW

    @functools.partial(
        pl.kernel,
        out_type=(
            jax.ShapeDtypeStruct((p_tot, d_model), jnp.float32),
            jax.ShapeDtypeStruct((p_tot, WLANES), jnp.float32),
        ),
        mesh=_sc_mesh(),
        scratch_types=[
            pltpu.VMEM((bpw,), jnp.int32),
            pltpu.VMEM((bpw, d_model), jnp.float32),
            pltpu.VMEM((bpw, WLANES), jnp.float32),
            pltpu.SemaphoreType.DMA,
            pltpu.SemaphoreType.DMA,
        ],
    )
    def dispatch(x_hbm, wrow_hbm, inv_hbm, xs_hbm, ws_hbm, idx_v, rows_v, wrows_v, sem_a, sem_b):
        wid = lax.axis_index("s") * NC + lax.axis_index("c")
        base = wid * bpw
        pltpu.sync_copy(inv_hbm.at[pl.ds(base, bpw)], idx_v)
        pltpu.sync_copy(x_hbm.at[pl.ds(base, bpw)], rows_v)
        pltpu.sync_copy(wrow_hbm.at[pl.ds(base, bpw)], wrows_v)
        cp_a = pltpu.async_copy(rows_v, xs_hbm.at[idx_v], sem_a)
        cp_b = pltpu.async_copy(wrows_v, ws_hbm.at[idx_v], sem_b)
        cp_a.wait()
        cp_b.wait()

    return dispatch


def _sc_combine_fn(n_tokens, p_tot, d_model):
    bpw = n_tokens // NW

    @functools.partial(
        pl.kernel,
        out_type=jax.ShapeDtypeStruct((n_tokens, d_model), jnp.float32),
        mesh=_sc_mesh(),
        scratch_types=[
            pltpu.VMEM((bpw,), jnp.int32),
            pltpu.VMEM((bpw, d_model), jnp.float32),
            pltpu.SemaphoreType.DMA,
        ],
    )
    def combine(ys_hbm, inv_hbm, out_hbm, idx_v, rows_v, sem):
        wid = lax.axis_index("s") * NC + lax.axis_index("c")
        base = wid * bpw
        pltpu.sync_copy(inv_hbm.at[pl.ds(base, bpw)], idx_v)
        pltpu.async_copy(ys_hbm.at[idx_v], rows_v, sem).wait()
        pltpu.sync_copy(rows_v, out_hbm.at[pl.ds(base, bpw)])

    return combine


@jax.jit
def kernel(x, expert_indices, expert_weights, w1, w2, w3):
    n_tokens, d_model = x.shape
    n_exp = w1.shape[0]
    d_ff = w1.shape[2]

    p_tot = n_tokens + n_exp * (ALIGN - 1) + BLK
    p_tot = ((p_tot + BLK - 1) // BLK) * BLK

    e_col = expert_indices
    if e_col.dtype != jnp.int32:
        e_col = e_col.astype(jnp.int32)
    if e_col.shape != (n_tokens, 1):
        e_col = e_col.reshape(n_tokens, 1)
    w_col = expert_weights
    if w_col.dtype != jnp.float32:
        w_col = w_col.astype(jnp.float32)
    if w_col.shape != (n_tokens, 1):
        w_col = w_col.reshape(n_tokens, 1)

    inv2d, off2d, nblk2d, wrow = pl.pallas_call(
        _meta_body,
        out_shape=(
            jax.ShapeDtypeStruct((n_tokens, 1), jnp.int32),
            jax.ShapeDtypeStruct((1, n_exp), jnp.int32),
            jax.ShapeDtypeStruct((1, n_exp), jnp.int32),
            jax.ShapeDtypeStruct((n_tokens, WLANES), jnp.float32),
        ),
    )(e_col, w_col)
    inv = inv2d.reshape(n_tokens)

    xs, ws = _sc_dispatch_fn(n_tokens, p_tot, d_model)(x, wrow, inv)

    ys = pl.pallas_call(
        _ffn_body,
        grid=(n_exp // EPG,),
        in_specs=[
            pl.BlockSpec(memory_space=pltpu.SMEM),
            pl.BlockSpec(memory_space=pltpu.SMEM),
            pl.BlockSpec((p_tot, d_model), lambda e: (0, 0)),
            pl.BlockSpec((p_tot, WLANES), lambda e: (0, 0)),
            pl.BlockSpec((EPG, d_model, d_ff), lambda e: (e, 0, 0)),
            pl.BlockSpec((EPG, d_model, d_ff), lambda e: (e, 0, 0)),
            pl.BlockSpec((EPG, d_ff, d_model), lambda e: (e, 0, 0)),
        ],
        out_specs=pl.BlockSpec((p_tot, d_model), lambda e: (0, 0)),
        out_shape=jax.ShapeDtypeStruct((p_tot, d_model), jnp.float32),
        compiler_params=pltpu.CompilerParams(
            dimension_semantics=("arbitrary",),
        ),
    )(off2d, nblk2d, xs, ws, w1, w2, w3)

    return _sc_combine_fn(n_tokens, p_tot, d_model)(ys, inv)
